# Initial kernel scaffold; baseline (speedup 1.0000x reference)
import jax
import jax.numpy as jnp
from jax.experimental import pallas as pl


def kernel(hidden_states, gate_weight, e_score_correction_bias, expert_gate_w, expert_up_w, expert_down_w, shared_gate_w, shared_up_w, shared_down_w):
    raise NotImplementedError("write your pallas kernel here")


# speedup vs baseline: 1.4203x; 1.4203x over previous
import functools

import jax
import jax.numpy as jnp
from jax.experimental import pallas as pl
from jax.experimental.pallas import tpu as pltpu

HID = 1024
INT = 512
NE = 8
TOP_SCALE = 2.5
TOK_TILE = 256


def _sigmoid(x):
    return 1.0 / (1.0 + jnp.exp(-x))


def _silu(x):
    return x * _sigmoid(x)


def _dotT(a, b):
    return jax.lax.dot_general(a, b, (((1,), (1,)), ((), ())),
                               preferred_element_type=jnp.float32)


def _router_comb(x, gate_w):
    logits = _dotT(x, gate_w)
    s = _sigmoid(logits)
    ei = jax.lax.broadcasted_iota(jnp.int32, s.shape, 1)
    m1 = jnp.max(s, axis=1, keepdims=True)
    f1 = jnp.min(jnp.where(s == m1, ei, NE), axis=1, keepdims=True)
    mask1 = ei == f1
    s2 = jnp.where(mask1, -1.0, s)
    m2 = jnp.max(s2, axis=1, keepdims=True)
    f2 = jnp.min(jnp.where(s2 == m2, ei, NE), axis=1, keepdims=True)
    mask2 = ei == f2
    scale = TOP_SCALE / (m1 + m2 + 1e-20)
    return (jnp.where(mask1, m1, 0.0) + jnp.where(mask2, m2, 0.0)) * scale


def _dense_body(x_ref, gw_ref, egw_ref, euw_ref, edw_ref,
                sgw_ref, suw_ref, sdw_ref, out_ref, acc_ref):
    e = pl.program_id(0)
    t = pl.program_id(1)
    x = x_ref[...]

    comb = _router_comb(x, gw_ref[...])
    ei = jax.lax.broadcasted_iota(jnp.int32, comb.shape, 1)
    c_col = jnp.sum(jnp.where(ei == e, comb, 0.0), axis=1, keepdims=True)

    g = _dotT(x, egw_ref[0])
    u = _dotT(x, euw_ref[0])
    h = _silu(g) * u
    o = _dotT(h, edw_ref[0]) * c_col

    rows = pl.ds(t * TOK_TILE, TOK_TILE)

    @pl.when(e == 0)
    def _init():
        sg = _dotT(x, sgw_ref[...])
        su = _dotT(x, suw_ref[...])
        shared = _dotT(_silu(sg) * su, sdw_ref[...])
        acc_ref[rows, :] = shared + o

    @pl.when(e > 0)
    def _accum():
        acc_ref[rows, :] += o

    @pl.when(e == NE - 1)
    def _emit():
        out_ref[...] = acc_ref[rows, :]


@jax.jit
def _moe(x2d, gate_weight, expert_gate_w, expert_up_w, expert_down_w,
         shared_gate_w, shared_up_w, shared_down_w):
    T = x2d.shape[0]
    grid = (NE, T // TOK_TILE)
    full = lambda shape: pl.BlockSpec(shape, lambda e, t: (0,) * len(shape))
    return pl.pallas_call(
        _dense_body,
        grid=grid,
        in_specs=[
            pl.BlockSpec((TOK_TILE, HID), lambda e, t: (t, 0)),
            full((NE, HID)),
            pl.BlockSpec((1, INT, HID), lambda e, t: (e, 0, 0)),
            pl.BlockSpec((1, INT, HID), lambda e, t: (e, 0, 0)),
            pl.BlockSpec((1, HID, INT), lambda e, t: (e, 0, 0)),
            full((INT, HID)),
            full((INT, HID)),
            full((HID, INT)),
        ],
        out_specs=pl.BlockSpec((TOK_TILE, HID), lambda e, t: (t, 0)),
        out_shape=jax.ShapeDtypeStruct((T, HID), jnp.float32),
        scratch_shapes=[pltpu.VMEM((T, HID), jnp.float32)],
    )(x2d, gate_weight, expert_gate_w, expert_up_w, expert_down_w,
      shared_gate_w, shared_up_w, shared_down_w)


def kernel(hidden_states, gate_weight, e_score_correction_bias,
           expert_gate_w, expert_up_w, expert_down_w,
           shared_gate_w, shared_up_w, shared_down_w):
    orig_shape = hidden_states.shape
    x2d = hidden_states.reshape(-1, orig_shape[-1])
    out = _moe(x2d, gate_weight, expert_gate_w, expert_up_w, expert_down_w,
               shared_gate_w, shared_up_w, shared_down_w)
    return out.reshape(orig_shape)


# speedup vs baseline: 1.6926x; 1.1917x over previous
import functools

import jax
import jax.numpy as jnp
from jax.experimental import pallas as pl
from jax.experimental.pallas import tpu as pltpu

HID = 1024
INT = 512
NE = 8
TOP_SCALE = 2.5
TOK_TILE = 256


def _sigmoid(x):
    return 1.0 / (1.0 + jnp.exp(-x))


def _silu(x):
    return x * _sigmoid(x)


def _dotT(a, b):
    return jax.lax.dot_general(a, b, (((1,), (1,)), ((), ())),
                               preferred_element_type=jnp.float32)


def _router_comb(x, gate_w):
    logits = _dotT(x, gate_w)
    s = _sigmoid(logits)
    ei = jax.lax.broadcasted_iota(jnp.int32, s.shape, 1)
    m1 = jnp.max(s, axis=1, keepdims=True)
    f1 = jnp.min(jnp.where(s == m1, ei, NE), axis=1, keepdims=True)
    mask1 = ei == f1
    s2 = jnp.where(mask1, -1.0, s)
    m2 = jnp.max(s2, axis=1, keepdims=True)
    f2 = jnp.min(jnp.where(s2 == m2, ei, NE), axis=1, keepdims=True)
    mask2 = ei == f2
    scale = TOP_SCALE / (m1 + m2 + 1e-20)
    return (jnp.where(mask1, m1, 0.0) + jnp.where(mask2, m2, 0.0)) * scale


def _dense_body(x_ref, gw_ref, egw_ref, euw_ref, edw_ref,
                sgw_ref, suw_ref, sdw_ref, out_ref):
    x = x_ref[...]
    comb = _router_comb(x, gw_ref[...])
    xb = x.astype(jnp.bfloat16)

    sg = _dotT(xb, sgw_ref[...])
    su = _dotT(xb, suw_ref[...])
    acc = _dotT((_silu(sg) * su).astype(jnp.bfloat16), sdw_ref[...])

    for e in range(NE):
        g = _dotT(xb, egw_ref[e])
        u = _dotT(xb, euw_ref[e])
        h = _silu(g) * u
        o = _dotT(h.astype(jnp.bfloat16), edw_ref[e])
        acc = acc + o * comb[:, e:e + 1]

    out_ref[...] = acc


@jax.jit
def _moe(x2d, gate_weight, expert_gate_w, expert_up_w, expert_down_w,
         shared_gate_w, shared_up_w, shared_down_w):
    T = x2d.shape[0]
    grid = (T // TOK_TILE,)
    full = lambda shape: pl.BlockSpec(shape, lambda t: (0,) * len(shape))
    return pl.pallas_call(
        _dense_body,
        grid=grid,
        in_specs=[
            pl.BlockSpec((TOK_TILE, HID), lambda t: (t, 0)),
            full((NE, HID)),
            full((NE, INT, HID)),
            full((NE, INT, HID)),
            full((NE, HID, INT)),
            full((INT, HID)),
            full((INT, HID)),
            full((HID, INT)),
        ],
        out_specs=pl.BlockSpec((TOK_TILE, HID), lambda t: (t, 0)),
        out_shape=jax.ShapeDtypeStruct((T, HID), jnp.float32),
    )(x2d, gate_weight, expert_gate_w, expert_up_w, expert_down_w,
      shared_gate_w, shared_up_w, shared_down_w)


def kernel(hidden_states, gate_weight, e_score_correction_bias,
           expert_gate_w, expert_up_w, expert_down_w,
           shared_gate_w, shared_up_w, shared_down_w):
    orig_shape = hidden_states.shape
    x2d = hidden_states.reshape(-1, orig_shape[-1])
    bf = jnp.bfloat16
    out = _moe(x2d, gate_weight,
               expert_gate_w.astype(bf), expert_up_w.astype(bf),
               expert_down_w.astype(bf),
               shared_gate_w.astype(bf), shared_up_w.astype(bf),
               shared_down_w.astype(bf))
    return out.reshape(orig_shape)
